# baseline (speedup 1.0000x reference)
import functools

import jax
import jax.numpy as jnp
from jax import lax
from jax.experimental import pallas as pl
from jax.experimental.pallas import tpu as pltpu
from jax.experimental.pallas import tpu_sc as plsc

N = 10000
E = 320000
H = 128

EDGE_BLK = 2000

_NC, _NS = 2, 16
_NW = _NC * _NS
_EPW = E // _NW
_GC = 80
_NPS = N // _NS
_EPC = E // _NC


def _sc_mesh():
    return plsc.VectorSubcoreMesh(core_axis_name="c", subcore_axis_name="s",
                                  num_cores=_NC, num_subcores=_NS)


def _sc_gather2(pre_dst, pre_src, dst, src):
    out_t = (jax.ShapeDtypeStruct((E, H), jnp.float32),
             jax.ShapeDtypeStruct((E, H), jnp.float32))

    @functools.partial(
        pl.kernel, out_type=out_t, mesh=_sc_mesh(),
        scratch_types=[
            pltpu.VMEM((_GC,), jnp.int32),
            pltpu.VMEM((_GC,), jnp.int32),
            pltpu.VMEM((_GC, H), jnp.float32),
            pltpu.VMEM((_GC, H), jnp.float32),
            pltpu.SemaphoreType.DMA,
            pltpu.SemaphoreType.DMA,
        ])
    def gk(pd_hbm, ps_hbm, dst_hbm, src_hbm, g1_hbm, g2_hbm,
           idx1, idx2, r1, r2, s1, s2):
        wid = lax.axis_index("s") * _NC + lax.axis_index("c")
        base = wid * _EPW

        @pl.loop(0, _EPW // _GC)
        def _(j):
            off = base + j * _GC
            pltpu.sync_copy(dst_hbm.at[pl.ds(off, _GC)], idx1)
            pltpu.sync_copy(src_hbm.at[pl.ds(off, _GC)], idx2)
            c1 = pltpu.async_copy(pd_hbm.at[idx1], r1, s1)
            c2 = pltpu.async_copy(ps_hbm.at[idx2], r2, s2)
            c1.wait()
            c2.wait()
            pltpu.sync_copy(r1, g1_hbm.at[pl.ds(off, _GC)])
            pltpu.sync_copy(r2, g2_hbm.at[pl.ds(off, _GC)])

    return gk(pre_dst, pre_src, dst, src)


def _sc_segsum(upd_e, dst, zeros):
    out_t = jax.ShapeDtypeStruct((_NC, N, H), jnp.float32)

    @functools.partial(
        pl.kernel, out_type=out_t, mesh=_sc_mesh(),
        scratch_types=[
            pltpu.VMEM((_GC,), jnp.int32),
            pltpu.VMEM((_GC, H), jnp.float32),
            pltpu.VMEM_SHARED((N, H), jnp.float32),
        ])
    def sk(ue_hbm, dst_hbm, z_hbm, out_hbm, idx, data, acc):
        cid = lax.axis_index("c")
        sid = lax.axis_index("s")

        @pl.loop(sid, N // _GC, step=_NS)
        def _(c):
            off = c * _GC
            pltpu.sync_copy(z_hbm.at[pl.ds(off, _GC)],
                            acc.at[pl.ds(off, _GC)])

        plsc.subcore_barrier()
        base = cid * _EPC + sid * _EPW

        @pl.loop(0, _EPW // _GC)
        def _(j):
            off = base + j * _GC
            pltpu.sync_copy(dst_hbm.at[pl.ds(off, _GC)], idx)
            pltpu.sync_copy(ue_hbm.at[pl.ds(off, _GC)], data)
            pltpu.sync_copy(data, acc.at[idx], add=True)

        plsc.subcore_barrier()

        @pl.loop(sid, N // _GC, step=_NS)
        def _(c):
            off = c * _GC
            pltpu.sync_copy(acc.at[pl.ds(off, _GC)],
                            out_hbm.at[cid, pl.ds(off, _GC)])

    return sk(upd_e, dst, zeros)


def _ln(t, g, beta):
    mu = jnp.mean(t, axis=-1, keepdims=True)
    var = jnp.mean((t - mu) ** 2, axis=-1, keepdims=True)
    return (t - mu) * jax.lax.rsqrt(var + 1e-5) * g + beta


def _edge_encoder_body(ea_ref, mean_ref, std_ref, w1_ref, b1_ref, w2_ref,
                       b2_ref, g_ref, beta_ref, out_ref):
    en = (ea_ref[...] - mean_ref[...]) / std_ref[...]
    h1 = jnp.maximum(jnp.dot(en, w1_ref[...],
                             preferred_element_type=jnp.float32) + b1_ref[...],
                     0.0)
    t = jnp.dot(h1, w2_ref[...], preferred_element_type=jnp.float32) + b2_ref[...]
    out_ref[...] = _ln(t, g_ref[...], beta_ref[...])


def _edge_mlp_body(e_ref, g1_ref, g2_ref, w1e_ref, b1_ref, w2_ref, b2_ref,
                   g_ref, beta_ref, out_ref):
    e = e_ref[...]
    h1 = jnp.maximum(
        jnp.dot(e, w1e_ref[...], preferred_element_type=jnp.float32)
        + g1_ref[...] + g2_ref[...] + b1_ref[...], 0.0)
    t = jnp.dot(h1, w2_ref[...], preferred_element_type=jnp.float32) + b2_ref[...]
    out_ref[...] = _ln(t, g_ref[...], beta_ref[...]) + e


def _edge_grid_call(body, n_in_edge_arrays, edge_arrays, small_arrays,
                    out_dim=H):
    grid = (E // EDGE_BLK,)
    in_specs = []
    for a in edge_arrays:
        in_specs.append(pl.BlockSpec((EDGE_BLK, a.shape[1]), lambda i: (i, 0)))
    for a in small_arrays:
        in_specs.append(pl.BlockSpec(a.shape, lambda i: (0,) * a.ndim))
    return pl.pallas_call(
        body,
        grid=grid,
        in_specs=in_specs,
        out_specs=pl.BlockSpec((EDGE_BLK, out_dim), lambda i: (i, 0)),
        out_shape=jax.ShapeDtypeStruct((E, out_dim), jnp.float32),
    )(*edge_arrays, *small_arrays)


def _node_encoder_body(x_ref, mean_ref, std_ref, w1_ref, b1_ref, w2_ref,
                       b2_ref, g_ref, beta_ref, out_ref):
    xn = (x_ref[...] - mean_ref[...]) / std_ref[...]
    h1 = jnp.maximum(jnp.dot(xn, w1_ref[...],
                             preferred_element_type=jnp.float32) + b1_ref[...],
                     0.0)
    t = jnp.dot(h1, w2_ref[...], preferred_element_type=jnp.float32) + b2_ref[...]
    out_ref[...] = _ln(t, g_ref[...], beta_ref[...])


def _node_update_body(h_ref, agg1_ref, agg2_ref, w1h_ref, w1a_ref, b1_ref,
                      w2_ref, b2_ref, g_ref, beta_ref, out_ref):
    h = h_ref[...]
    agg = agg1_ref[...] + agg2_ref[...]
    h1 = jnp.maximum(
        jnp.dot(h, w1h_ref[...], preferred_element_type=jnp.float32)
        + jnp.dot(agg, w1a_ref[...], preferred_element_type=jnp.float32)
        + b1_ref[...], 0.0)
    t = jnp.dot(h1, w2_ref[...], preferred_element_type=jnp.float32) + b2_ref[...]
    out_ref[...] = h + _ln(t, g_ref[...], beta_ref[...])


def _pre_body(h_ref, wd_ref, ws_ref, pd_ref, ps_ref):
    h = h_ref[...]
    pd_ref[...] = jnp.dot(h, wd_ref[...], preferred_element_type=jnp.float32)
    ps_ref[...] = jnp.dot(h, ws_ref[...], preferred_element_type=jnp.float32)


def _decoder_body(h_ref, w1_ref, b1_ref, w2_ref, b2_ref, out_ref):
    h1 = jnp.maximum(jnp.dot(h_ref[...], w1_ref[...],
                             preferred_element_type=jnp.float32) + b1_ref[...],
                     0.0)
    out_ref[...] = (jnp.dot(h1, w2_ref[...], preferred_element_type=jnp.float32)
                    + b2_ref[...])


def _whole_call(body, arrays, out_shapes):
    in_specs = [pl.BlockSpec(a.shape, lambda: (0,) * a.ndim) for a in arrays]
    if isinstance(out_shapes[0], tuple):
        out_specs = tuple(pl.BlockSpec(s, lambda: (0,) * len(s))
                          for s in out_shapes)
        out_shape = tuple(jax.ShapeDtypeStruct(s, jnp.float32)
                          for s in out_shapes)
    else:
        out_specs = pl.BlockSpec(out_shapes, lambda: (0,) * len(out_shapes))
        out_shape = jax.ShapeDtypeStruct(out_shapes, jnp.float32)
    return pl.pallas_call(body, in_specs=in_specs, out_specs=out_specs,
                          out_shape=out_shape)(*arrays)


def _row(v):
    return v.reshape(1, -1)


def kernel(x, edge_index, edge_attr, mean_vec_x, std_vec_x, mean_vec_edge,
           std_vec_edge, params):
    p = params
    src = edge_index[0]
    dst = edge_index[1]
    zeros_nh = jnp.zeros((N, H), jnp.float32)

    h = _whole_call(
        _node_encoder_body,
        (x, _row(mean_vec_x), _row(std_vec_x), p['enc_node_W1'],
         _row(p['enc_node_b1']), p['enc_node_W2'], _row(p['enc_node_b2']),
         _row(p['enc_node_g']), _row(p['enc_node_beta'])),
        (N, H))

    e = _edge_grid_call(
        _edge_encoder_body, 1,
        (edge_attr,),
        (_row(mean_vec_edge), _row(std_vec_edge), p['enc_edge_W1'],
         _row(p['enc_edge_b1']), p['enc_edge_W2'], _row(p['enc_edge_b2']),
         _row(p['enc_edge_g']), _row(p['enc_edge_beta'])))

    for i in range(4):
        w1 = p['proc_edge_W1'][i]
        w1_dst, w1_src, w1_e = w1[:H], w1[H:2 * H], w1[2 * H:]

        pre_dst, pre_src = _whole_call(_pre_body, (h, w1_dst, w1_src),
                                       ((N, H), (N, H)))

        g1, g2 = _sc_gather2(pre_dst, pre_src, dst, src)

        upd_e = _edge_grid_call(
            _edge_mlp_body, 3,
            (e, g1, g2),
            (w1_e, _row(p['proc_edge_b1'][i]), p['proc_edge_W2'][i],
             _row(p['proc_edge_b2'][i]), _row(p['proc_edge_g'][i]),
             _row(p['proc_edge_beta'][i])))

        aggp = _sc_segsum(upd_e, dst, zeros_nh)

        nw1 = p['proc_node_W1'][i]
        h = _whole_call(
            _node_update_body,
            (h, aggp[0], aggp[1], nw1[:H], nw1[H:], _row(p['proc_node_b1'][i]),
             p['proc_node_W2'][i], _row(p['proc_node_b2'][i]),
             _row(p['proc_node_g'][i]), _row(p['proc_node_beta'][i])),
            (N, H))
        e = upd_e

    out = _whole_call(
        _decoder_body,
        (h, p['dec_W1'], _row(p['dec_b1']), p['dec_W2'], _row(p['dec_b2'])),
        (N, 1))
    return out


# speedup vs baseline: 1.5122x; 1.5122x over previous
import functools

import jax
import jax.numpy as jnp
from jax import lax
from jax.experimental import pallas as pl
from jax.experimental.pallas import tpu as pltpu
from jax.experimental.pallas import tpu_sc as plsc

N = 10000
E = 320000
H = 128

EDGE_BLK = 2000

_NC, _NS = 2, 16
_NW = _NC * _NS
_EPW = E // _NW
_GC = 80
_NPS = N // _NS
_EPC = E // _NC


def _sc_mesh():
    return plsc.VectorSubcoreMesh(core_axis_name="c", subcore_axis_name="s",
                                  num_cores=_NC, num_subcores=_NS)


_ES = E // _NS
_IB = 4000
_NBLK = _ES // _IB
_CPB = _IB // _GC


def _sc_gather2(pre_dst, pre_src, dst, src):
    out_t = (jax.ShapeDtypeStruct((E, H), jnp.float32),
             jax.ShapeDtypeStruct((E, H), jnp.float32))

    @functools.partial(
        pl.kernel, out_type=out_t, mesh=_sc_mesh(),
        scratch_types=[
            pltpu.VMEM((_IB,), jnp.int32),
            pltpu.VMEM((_GC, H), jnp.float32),
            pltpu.VMEM((_GC, H), jnp.float32),
            pltpu.VMEM_SHARED((N, H), jnp.float32),
            pltpu.SemaphoreType.DMA,
            pltpu.SemaphoreType.DMA,
            pltpu.SemaphoreType.DMA,
        ])
    def gk(pd_hbm, ps_hbm, dst_hbm, src_hbm, g1_hbm, g2_hbm,
           idxb, r0, r1, tab, gsem, wb0, wb1):
        cid = lax.axis_index("c")
        sid = lax.axis_index("s")

        def one_table(tab_hbm, i_hbm, out_hbm):
            @pl.loop(sid, N // _GC, step=_NS)
            def _(ci):
                off = ci * _GC
                pltpu.sync_copy(tab_hbm.at[pl.ds(off, _GC)],
                                tab.at[pl.ds(off, _GC)])

            plsc.subcore_barrier()
            base = sid * _ES
            rows = (r0, r1)
            wbs = (wb0, wb1)

            for blk in range(_NBLK):
                bbase = base + blk * _IB
                pltpu.sync_copy(i_hbm.at[pl.ds(bbase, _IB)], idxb)

                @pl.loop(0, _CPB // 2)
                def _(jj):
                    for b in range(2):
                        j = jj * 2 + b
                        ioff = j * _GC
                        ooff = bbase + ioff
                        if blk == 0:
                            @pl.when(jj > 0)
                            def _():
                                pltpu.make_async_copy(
                                    rows[b],
                                    out_hbm.at[pl.ds(ooff, _GC)],
                                    wbs[b]).wait()
                        else:
                            pltpu.make_async_copy(
                                rows[b], out_hbm.at[pl.ds(ooff, _GC)],
                                wbs[b]).wait()
                        pltpu.async_copy(
                            tab.at[idxb.at[pl.ds(ioff, _GC)]],
                            rows[b], gsem).wait()
                        pltpu.async_copy(rows[b],
                                         out_hbm.at[pl.ds(ooff, _GC)],
                                         wbs[b])

            for b in range(2):
                pltpu.make_async_copy(rows[b], out_hbm.at[pl.ds(0, _GC)],
                                      wbs[b]).wait()

        @pl.when(cid == 0)
        def _():
            one_table(pd_hbm, dst_hbm, g1_hbm)

        @pl.when(cid == 1)
        def _():
            one_table(ps_hbm, src_hbm, g2_hbm)

    return gk(pre_dst, pre_src, dst, src)


def _sc_segsum(upd_e, dst, zeros):
    out_t = jax.ShapeDtypeStruct((_NC, N, H), jnp.float32)

    nchunks = _EPW // _GC

    @functools.partial(
        pl.kernel, out_type=out_t, mesh=_sc_mesh(),
        scratch_types=[
            pltpu.VMEM((_GC,), jnp.int32),
            pltpu.VMEM((_GC,), jnp.int32),
            pltpu.VMEM((_GC, H), jnp.float32),
            pltpu.VMEM((_GC, H), jnp.float32),
            pltpu.VMEM_SHARED((N, H), jnp.float32),
            pltpu.SemaphoreType.DMA,
            pltpu.SemaphoreType.DMA,
            pltpu.SemaphoreType.DMA,
            pltpu.SemaphoreType.DMA,
        ])
    def sk(ue_hbm, dst_hbm, z_hbm, out_hbm, i0, i1, d0, d1, acc,
           is0, is1, ds0, ds1):
        cid = lax.axis_index("c")
        sid = lax.axis_index("s")

        @pl.loop(sid, N // _GC, step=_NS)
        def _(c):
            off = c * _GC
            pltpu.sync_copy(z_hbm.at[pl.ds(off, _GC)],
                            acc.at[pl.ds(off, _GC)])

        base = cid * _EPC + sid * _EPW
        plsc.subcore_barrier()

        idxs = (i0, i1)
        data = (d0, d1)
        isems = (is0, is1)
        dsems = (ds0, ds1)

        def load(c, slot):
            off = base + c * _GC
            pltpu.async_copy(dst_hbm.at[pl.ds(off, _GC)], idxs[slot],
                             isems[slot])
            pltpu.async_copy(ue_hbm.at[pl.ds(off, _GC)], data[slot],
                             dsems[slot])

        def wait_loaded(c, slot):
            off = base + c * _GC
            pltpu.make_async_copy(dst_hbm.at[pl.ds(off, _GC)], idxs[slot],
                                  isems[slot]).wait()
            pltpu.make_async_copy(ue_hbm.at[pl.ds(off, _GC)], data[slot],
                                  dsems[slot]).wait()

        load(0, 0)

        @pl.loop(0, (nchunks - 1) // 2)
        def _(jj):
            for b in range(2):
                c = jj * 2 + b
                wait_loaded(c, b)

                @pl.when(c + 1 < nchunks)
                def _():
                    load(c + 1, 1 - b)

                pltpu.sync_copy(data[b], acc.at[idxs[b]], add=True)

        wait_loaded(nchunks - 1, 0)
        pltpu.sync_copy(data[0], acc.at[idxs[0]], add=True)

        plsc.subcore_barrier()

        @pl.loop(sid, N // _GC, step=_NS)
        def _(c):
            off = c * _GC
            pltpu.sync_copy(acc.at[pl.ds(off, _GC)],
                            out_hbm.at[cid, pl.ds(off, _GC)])

    return sk(upd_e, dst, zeros)


def _ln(t, g, beta):
    mu = jnp.mean(t, axis=-1, keepdims=True)
    var = jnp.mean((t - mu) ** 2, axis=-1, keepdims=True)
    return (t - mu) * jax.lax.rsqrt(var + 1e-5) * g + beta


def _edge_encoder_body(ea_ref, mean_ref, std_ref, w1_ref, b1_ref, w2_ref,
                       b2_ref, g_ref, beta_ref, out_ref):
    en = (ea_ref[...] - mean_ref[...]) / std_ref[...]
    h1 = jnp.maximum(jnp.dot(en, w1_ref[...],
                             preferred_element_type=jnp.float32) + b1_ref[...],
                     0.0)
    t = jnp.dot(h1, w2_ref[...], preferred_element_type=jnp.float32) + b2_ref[...]
    out_ref[...] = _ln(t, g_ref[...], beta_ref[...])


def _edge_mlp_body(e_ref, g1_ref, g2_ref, w1e_ref, b1_ref, w2_ref, b2_ref,
                   g_ref, beta_ref, out_ref):
    e = e_ref[...]
    h1 = jnp.maximum(
        jnp.dot(e, w1e_ref[...], preferred_element_type=jnp.float32)
        + g1_ref[...] + g2_ref[...] + b1_ref[...], 0.0)
    t = jnp.dot(h1, w2_ref[...], preferred_element_type=jnp.float32) + b2_ref[...]
    out_ref[...] = _ln(t, g_ref[...], beta_ref[...]) + e


def _edge_grid_call(body, n_in_edge_arrays, edge_arrays, small_arrays,
                    out_dim=H):
    grid = (E // EDGE_BLK,)
    in_specs = []
    for a in edge_arrays:
        in_specs.append(pl.BlockSpec((EDGE_BLK, a.shape[1]), lambda i: (i, 0)))
    for a in small_arrays:
        in_specs.append(pl.BlockSpec(a.shape, lambda i: (0,) * a.ndim))
    return pl.pallas_call(
        body,
        grid=grid,
        in_specs=in_specs,
        out_specs=pl.BlockSpec((EDGE_BLK, out_dim), lambda i: (i, 0)),
        out_shape=jax.ShapeDtypeStruct((E, out_dim), jnp.float32),
    )(*edge_arrays, *small_arrays)


def _node_encoder_body(x_ref, mean_ref, std_ref, w1_ref, b1_ref, w2_ref,
                       b2_ref, g_ref, beta_ref, out_ref):
    xn = (x_ref[...] - mean_ref[...]) / std_ref[...]
    h1 = jnp.maximum(jnp.dot(xn, w1_ref[...],
                             preferred_element_type=jnp.float32) + b1_ref[...],
                     0.0)
    t = jnp.dot(h1, w2_ref[...], preferred_element_type=jnp.float32) + b2_ref[...]
    out_ref[...] = _ln(t, g_ref[...], beta_ref[...])


def _node_update_body(h_ref, agg1_ref, agg2_ref, w1h_ref, w1a_ref, b1_ref,
                      w2_ref, b2_ref, g_ref, beta_ref, out_ref):
    h = h_ref[...]
    agg = agg1_ref[...] + agg2_ref[...]
    h1 = jnp.maximum(
        jnp.dot(h, w1h_ref[...], preferred_element_type=jnp.float32)
        + jnp.dot(agg, w1a_ref[...], preferred_element_type=jnp.float32)
        + b1_ref[...], 0.0)
    t = jnp.dot(h1, w2_ref[...], preferred_element_type=jnp.float32) + b2_ref[...]
    out_ref[...] = h + _ln(t, g_ref[...], beta_ref[...])


def _pre_body(h_ref, wd_ref, ws_ref, pd_ref, ps_ref):
    h = h_ref[...]
    pd_ref[...] = jnp.dot(h, wd_ref[...], preferred_element_type=jnp.float32)
    ps_ref[...] = jnp.dot(h, ws_ref[...], preferred_element_type=jnp.float32)


def _decoder_body(h_ref, w1_ref, b1_ref, w2_ref, b2_ref, out_ref):
    h1 = jnp.maximum(jnp.dot(h_ref[...], w1_ref[...],
                             preferred_element_type=jnp.float32) + b1_ref[...],
                     0.0)
    out_ref[...] = (jnp.dot(h1, w2_ref[...], preferred_element_type=jnp.float32)
                    + b2_ref[...])


def _whole_call(body, arrays, out_shapes):
    in_specs = [pl.BlockSpec(a.shape, lambda: (0,) * a.ndim) for a in arrays]
    if isinstance(out_shapes[0], tuple):
        out_specs = tuple(pl.BlockSpec(s, lambda: (0,) * len(s))
                          for s in out_shapes)
        out_shape = tuple(jax.ShapeDtypeStruct(s, jnp.float32)
                          for s in out_shapes)
    else:
        out_specs = pl.BlockSpec(out_shapes, lambda: (0,) * len(out_shapes))
        out_shape = jax.ShapeDtypeStruct(out_shapes, jnp.float32)
    return pl.pallas_call(body, in_specs=in_specs, out_specs=out_specs,
                          out_shape=out_shape)(*arrays)


def _row(v):
    return v.reshape(1, -1)


def kernel(x, edge_index, edge_attr, mean_vec_x, std_vec_x, mean_vec_edge,
           std_vec_edge, params):
    p = params
    src = edge_index[0]
    dst = edge_index[1]
    zeros_nh = jnp.zeros((N, H), jnp.float32)

    h = _whole_call(
        _node_encoder_body,
        (x, _row(mean_vec_x), _row(std_vec_x), p['enc_node_W1'],
         _row(p['enc_node_b1']), p['enc_node_W2'], _row(p['enc_node_b2']),
         _row(p['enc_node_g']), _row(p['enc_node_beta'])),
        (N, H))

    e = _edge_grid_call(
        _edge_encoder_body, 1,
        (edge_attr,),
        (_row(mean_vec_edge), _row(std_vec_edge), p['enc_edge_W1'],
         _row(p['enc_edge_b1']), p['enc_edge_W2'], _row(p['enc_edge_b2']),
         _row(p['enc_edge_g']), _row(p['enc_edge_beta'])))

    for i in range(4):
        w1 = p['proc_edge_W1'][i]
        w1_dst, w1_src, w1_e = w1[:H], w1[H:2 * H], w1[2 * H:]

        pre_dst, pre_src = _whole_call(_pre_body, (h, w1_dst, w1_src),
                                       ((N, H), (N, H)))

        g1, g2 = _sc_gather2(pre_dst, pre_src, dst, src)

        upd_e = _edge_grid_call(
            _edge_mlp_body, 3,
            (e, g1, g2),
            (w1_e, _row(p['proc_edge_b1'][i]), p['proc_edge_W2'][i],
             _row(p['proc_edge_b2'][i]), _row(p['proc_edge_g'][i]),
             _row(p['proc_edge_beta'][i])))

        aggp = _sc_segsum(upd_e, dst, zeros_nh)

        nw1 = p['proc_node_W1'][i]
        h = _whole_call(
            _node_update_body,
            (h, aggp[0], aggp[1], nw1[:H], nw1[H:], _row(p['proc_node_b1'][i]),
             p['proc_node_W2'][i], _row(p['proc_node_b2'][i]),
             _row(p['proc_node_g'][i]), _row(p['proc_node_beta'][i])),
            (N, H))
        e = upd_e

    out = _whole_call(
        _decoder_body,
        (h, p['dec_W1'], _row(p['dec_b1']), p['dec_W2'], _row(p['dec_b2'])),
        (N, 1))
    return out


# speedup vs baseline: 1.5556x; 1.0287x over previous
import functools

import jax
import jax.numpy as jnp
from jax import lax
from jax.experimental import pallas as pl
from jax.experimental.pallas import tpu as pltpu
from jax.experimental.pallas import tpu_sc as plsc

N = 10000
E = 320000
H = 128
E2 = E // 2

EDGE_BLK = 2000

_NC, _NS = 2, 16
_NW = _NC * _NS
_GC = 80
_SGC = 40


def _sc_mesh():
    return plsc.VectorSubcoreMesh(core_axis_name="c", subcore_axis_name="s",
                                  num_cores=_NC, num_subcores=_NS)


def _sc_gather2(pre_dst, pre_src, dst, src, ebase, ne):
    epw = ne // _NS
    nch = epw // _GC
    assert nch % 2 == 1 and epw % 8 == 0
    out_t = (jax.ShapeDtypeStruct((ne, H), jnp.float32),
             jax.ShapeDtypeStruct((ne, H), jnp.float32))

    @functools.partial(
        pl.kernel, out_type=out_t, mesh=_sc_mesh(),
        scratch_types=[
            pltpu.VMEM((epw,), jnp.int32),
            pltpu.VMEM((_GC, H), jnp.float32),
            pltpu.VMEM((_GC, H), jnp.float32),
            pltpu.VMEM_SHARED((N, H), jnp.float32),
            pltpu.SemaphoreType.DMA,
            pltpu.SemaphoreType.DMA,
            pltpu.SemaphoreType.DMA,
            pltpu.SemaphoreType.DMA,
        ])
    def gk(pd_hbm, ps_hbm, dst_hbm, src_hbm, g1_hbm, g2_hbm,
           idxw, r0, r1, tab, gs0, gs1, wb0, wb1):
        cid = lax.axis_index("c")
        sid = lax.axis_index("s")

        def one_table(tab_hbm, i_hbm, out_hbm):
            @pl.loop(sid, N // _GC, step=_NS)
            def _(ci):
                off = ci * _GC
                pltpu.sync_copy(tab_hbm.at[pl.ds(off, _GC)],
                                tab.at[pl.ds(off, _GC)])

            obase = sid * epw
            pltpu.sync_copy(i_hbm.at[pl.ds(ebase + obase, epw)], idxw)
            plsc.subcore_barrier()

            rows = (r0, r1)
            gsem = (gs0, gs1)
            wbs = (wb0, wb1)

            def gissue(c, b):
                pltpu.async_copy(tab.at[idxw.at[pl.ds(c * _GC, _GC)]],
                                 rows[b], gsem[b])

            def gwait(b):
                pltpu.make_async_copy(tab.at[idxw.at[pl.ds(0, _GC)]],
                                      rows[b], gsem[b]).wait()

            def wissue(c, b):
                pltpu.async_copy(rows[b],
                                 out_hbm.at[pl.ds(obase + c * _GC, _GC)],
                                 wbs[b])

            def wwait(b):
                pltpu.make_async_copy(rows[b], out_hbm.at[pl.ds(0, _GC)],
                                      wbs[b]).wait()

            gissue(0, 0)

            @pl.loop(0, (nch - 1) // 2)
            def _(jj):
                for b in range(2):
                    c = jj * 2 + b
                    gwait(b)
                    wissue(c, b)

                    @pl.when(c > 0)
                    def _():
                        wwait(1 - b)

                    @pl.when(c + 1 < nch)
                    def _():
                        gissue(c + 1, 1 - b)

            gwait(0)
            wissue(nch - 1, 0)
            wwait(1)
            wwait(0)

        @pl.when(cid == 0)
        def _():
            one_table(pd_hbm, dst_hbm, g1_hbm)

        @pl.when(cid == 1)
        def _():
            one_table(ps_hbm, src_hbm, g2_hbm)

    return gk(pre_dst, pre_src, dst, src)


def _sc_segsum(upd_e, dst, zeros, ebase, ne):
    epw = ne // _NW
    nch = epw // _SGC
    assert nch % 2 == 1 and epw % 8 == 0
    out_t = jax.ShapeDtypeStruct((_NC, N, H), jnp.float32)

    @functools.partial(
        pl.kernel, out_type=out_t, mesh=_sc_mesh(),
        scratch_types=[
            pltpu.VMEM((_SGC,), jnp.int32),
            pltpu.VMEM((_SGC,), jnp.int32),
            pltpu.VMEM((_SGC, H), jnp.float32),
            pltpu.VMEM((_SGC, H), jnp.float32),
            pltpu.VMEM_SHARED((N, H), jnp.float32),
            pltpu.SemaphoreType.DMA,
            pltpu.SemaphoreType.DMA,
            pltpu.SemaphoreType.DMA,
            pltpu.SemaphoreType.DMA,
        ])
    def sk(ue_hbm, dst_hbm, z_hbm, out_hbm, i0, i1, d0, d1, acc,
           is0, is1, ds0, ds1):
        cid = lax.axis_index("c")
        sid = lax.axis_index("s")

        @pl.loop(sid, N // _GC, step=_NS)
        def _(c):
            off = c * _GC
            pltpu.sync_copy(z_hbm.at[pl.ds(off, _GC)],
                            acc.at[pl.ds(off, _GC)])

        lbase = cid * (ne // 2) + sid * epw
        gbase = ebase + lbase
        plsc.subcore_barrier()

        idxs = (i0, i1)
        data = (d0, d1)
        isems = (is0, is1)
        dsems = (ds0, ds1)

        def load(c, slot):
            pltpu.async_copy(dst_hbm.at[pl.ds(gbase + c * _SGC, _SGC)],
                             idxs[slot], isems[slot])
            pltpu.async_copy(ue_hbm.at[pl.ds(lbase + c * _SGC, _SGC)],
                             data[slot], dsems[slot])

        def wait_loaded(slot):
            pltpu.make_async_copy(dst_hbm.at[pl.ds(gbase, _SGC)],
                                  idxs[slot], isems[slot]).wait()
            pltpu.make_async_copy(ue_hbm.at[pl.ds(lbase, _SGC)],
                                  data[slot], dsems[slot]).wait()

        load(0, 0)

        @pl.loop(0, (nch - 1) // 2)
        def _(jj):
            for b in range(2):
                c = jj * 2 + b
                wait_loaded(b)

                @pl.when(c + 1 < nch)
                def _():
                    load(c + 1, 1 - b)

                pltpu.sync_copy(data[b], acc.at[idxs[b]], add=True)

        wait_loaded(0)
        pltpu.sync_copy(data[0], acc.at[idxs[0]], add=True)

        plsc.subcore_barrier()

        @pl.loop(sid, N // _GC, step=_NS)
        def _(c):
            off = c * _GC
            pltpu.sync_copy(acc.at[pl.ds(off, _GC)],
                            out_hbm.at[cid, pl.ds(off, _GC)])

    return sk(upd_e, dst, zeros)


def _ln(t, g, beta):
    mu = jnp.mean(t, axis=-1, keepdims=True)
    var = jnp.mean((t - mu) ** 2, axis=-1, keepdims=True)
    return (t - mu) * jax.lax.rsqrt(var + 1e-5) * g + beta


def _edge_encoder_body(ea_ref, mean_ref, std_ref, w1_ref, b1_ref, w2_ref,
                       b2_ref, g_ref, beta_ref, out_ref):
    en = (ea_ref[...] - mean_ref[...]) / std_ref[...]
    h1 = jnp.maximum(jnp.dot(en, w1_ref[...],
                             preferred_element_type=jnp.float32) + b1_ref[...],
                     0.0)
    t = jnp.dot(h1, w2_ref[...], preferred_element_type=jnp.float32) + b2_ref[...]
    out_ref[...] = _ln(t, g_ref[...], beta_ref[...])


def _edge_mlp_body(e_ref, g1_ref, g2_ref, w1e_ref, b1_ref, w2_ref, b2_ref,
                   g_ref, beta_ref, out_ref):
    e = e_ref[...]
    h1 = jnp.maximum(
        jnp.dot(e, w1e_ref[...], preferred_element_type=jnp.float32)
        + g1_ref[...] + g2_ref[...] + b1_ref[...], 0.0)
    t = jnp.dot(h1, w2_ref[...], preferred_element_type=jnp.float32) + b2_ref[...]
    out_ref[...] = _ln(t, g_ref[...], beta_ref[...]) + e


def _node_encoder_body(x_ref, mean_ref, std_ref, w1_ref, b1_ref, w2_ref,
                       b2_ref, g_ref, beta_ref, out_ref):
    xn = (x_ref[...] - mean_ref[...]) / std_ref[...]
    h1 = jnp.maximum(jnp.dot(xn, w1_ref[...],
                             preferred_element_type=jnp.float32) + b1_ref[...],
                     0.0)
    t = jnp.dot(h1, w2_ref[...], preferred_element_type=jnp.float32) + b2_ref[...]
    out_ref[...] = _ln(t, g_ref[...], beta_ref[...])


def _node_update_body(h_ref, a1_ref, a2_ref, a3_ref, a4_ref, w1h_ref,
                      w1a_ref, b1_ref, w2_ref, b2_ref, g_ref, beta_ref,
                      out_ref):
    h = h_ref[...]
    agg = (a1_ref[...] + a2_ref[...]) + (a3_ref[...] + a4_ref[...])
    h1 = jnp.maximum(
        jnp.dot(h, w1h_ref[...], preferred_element_type=jnp.float32)
        + jnp.dot(agg, w1a_ref[...], preferred_element_type=jnp.float32)
        + b1_ref[...], 0.0)
    t = jnp.dot(h1, w2_ref[...], preferred_element_type=jnp.float32) + b2_ref[...]
    out_ref[...] = h + _ln(t, g_ref[...], beta_ref[...])


def _pre_body(h_ref, wd_ref, ws_ref, pd_ref, ps_ref):
    h = h_ref[...]
    pd_ref[...] = jnp.dot(h, wd_ref[...], preferred_element_type=jnp.float32)
    ps_ref[...] = jnp.dot(h, ws_ref[...], preferred_element_type=jnp.float32)


def _decoder_body(h_ref, w1_ref, b1_ref, w2_ref, b2_ref, out_ref):
    h1 = jnp.maximum(jnp.dot(h_ref[...], w1_ref[...],
                             preferred_element_type=jnp.float32) + b1_ref[...],
                     0.0)
    out_ref[...] = (jnp.dot(h1, w2_ref[...], preferred_element_type=jnp.float32)
                    + b2_ref[...])


def _edge_grid_call(body, edge_arrays, small_arrays, in_offset_rows=0,
                    n_rows=E2, out_dim=H):
    grid = (n_rows // EDGE_BLK,)
    off_blk = in_offset_rows // EDGE_BLK
    in_specs = []
    for k, a in enumerate(edge_arrays):
        full = a.shape[0] != n_rows
        off = off_blk if full else 0
        in_specs.append(pl.BlockSpec((EDGE_BLK, a.shape[1]),
                                     lambda i, off=off: (i + off, 0)))
    for a in small_arrays:
        in_specs.append(pl.BlockSpec(a.shape, lambda i: (0,) * a.ndim))
    return pl.pallas_call(
        body,
        grid=grid,
        in_specs=in_specs,
        out_specs=pl.BlockSpec((EDGE_BLK, out_dim), lambda i: (i, 0)),
        out_shape=jax.ShapeDtypeStruct((n_rows, out_dim), jnp.float32),
    )(*edge_arrays, *small_arrays)


def _whole_call(body, arrays, out_shapes):
    in_specs = [pl.BlockSpec(a.shape, lambda: (0,) * a.ndim) for a in arrays]
    if isinstance(out_shapes[0], tuple):
        out_specs = tuple(pl.BlockSpec(s, lambda: (0,) * len(s))
                          for s in out_shapes)
        out_shape = tuple(jax.ShapeDtypeStruct(s, jnp.float32)
                          for s in out_shapes)
    else:
        out_specs = pl.BlockSpec(out_shapes, lambda: (0,) * len(out_shapes))
        out_shape = jax.ShapeDtypeStruct(out_shapes, jnp.float32)
    return pl.pallas_call(body, in_specs=in_specs, out_specs=out_specs,
                          out_shape=out_shape)(*arrays)


def _row(v):
    return v.reshape(1, -1)


def kernel(x, edge_index, edge_attr, mean_vec_x, std_vec_x, mean_vec_edge,
           std_vec_edge, params):
    p = params
    src = edge_index[0]
    dst = edge_index[1]
    zeros_nh = jnp.zeros((N, H), jnp.float32)

    h = _whole_call(
        _node_encoder_body,
        (x, _row(mean_vec_x), _row(std_vec_x), p['enc_node_W1'],
         _row(p['enc_node_b1']), p['enc_node_W2'], _row(p['enc_node_b2']),
         _row(p['enc_node_g']), _row(p['enc_node_beta'])),
        (N, H))

    enc_small = (_row(mean_vec_edge), _row(std_vec_edge), p['enc_edge_W1'],
                 _row(p['enc_edge_b1']), p['enc_edge_W2'],
                 _row(p['enc_edge_b2']), _row(p['enc_edge_g']),
                 _row(p['enc_edge_beta']))
    e_h = [_edge_grid_call(_edge_encoder_body, (edge_attr,), enc_small,
                           in_offset_rows=j * E2) for j in range(2)]

    for i in range(4):
        w1 = p['proc_edge_W1'][i]
        w1_dst, w1_src, w1_e = w1[:H], w1[H:2 * H], w1[2 * H:]

        pre_dst, pre_src = _whole_call(_pre_body, (h, w1_dst, w1_src),
                                       ((N, H), (N, H)))

        edge_small = (w1_e, _row(p['proc_edge_b1'][i]), p['proc_edge_W2'][i],
                      _row(p['proc_edge_b2'][i]), _row(p['proc_edge_g'][i]),
                      _row(p['proc_edge_beta'][i]))

        g_h = [_sc_gather2(pre_dst, pre_src, dst, src, j * E2, E2)
               for j in range(2)]
        ue_h = [_edge_grid_call(_edge_mlp_body,
                                (e_h[j], g_h[j][0], g_h[j][1]), edge_small)
                for j in range(2)]
        aggs = [_sc_segsum(ue_h[j], dst, zeros_nh, j * E2, E2)
                for j in range(2)]

        nw1 = p['proc_node_W1'][i]
        h = _whole_call(
            _node_update_body,
            (h, aggs[0][0], aggs[0][1], aggs[1][0], aggs[1][1],
             nw1[:H], nw1[H:], _row(p['proc_node_b1'][i]),
             p['proc_node_W2'][i], _row(p['proc_node_b2'][i]),
             _row(p['proc_node_g'][i]), _row(p['proc_node_beta'][i])),
            (N, H))
        e_h = ue_h

    out = _whole_call(
        _decoder_body,
        (h, p['dec_W1'], _row(p['dec_b1']), p['dec_W2'], _row(p['dec_b2'])),
        (N, 1))
    return out
